# baseline (speedup 1.0000x reference)
import functools

import jax
import jax.numpy as jnp
from jax import lax
from jax.experimental import pallas as pl
from jax.experimental.pallas import tpu as pltpu
from jax.experimental.pallas import tpu_sc as plsc

B = 16384
NC = 2
NS = 16
NW = NC * NS
BPW = B // NW
CH = 8
NP = BPW // (2 * CH)

D_MLP = 32
D_MF = 16
BLK = 2048


def _sc_gather(user_idx, item_idx, U_mlp, I_mlp, U_mf, I_mf):
    mesh = plsc.VectorSubcoreMesh(core_axis_name="c", subcore_axis_name="s")

    @functools.partial(
        pl.kernel,
        mesh=mesh,
        compiler_params=pltpu.CompilerParams(use_tc_tiling_on_sc=True),
        out_type=jax.ShapeDtypeStruct((B, 128), jnp.float32),
        scratch_types=[
            pltpu.VMEM((BPW,), jnp.int32),
            pltpu.VMEM((BPW,), jnp.int32),
            pltpu.VMEM((CH * 8, D_MLP), jnp.float32),
            pltpu.VMEM((CH * 8, D_MLP), jnp.float32),
            pltpu.VMEM((CH * 8, D_MLP), jnp.float32),
            pltpu.VMEM((CH * 8, D_MLP), jnp.float32),
            pltpu.VMEM((CH * 8, D_MF), jnp.float32),
            pltpu.VMEM((CH * 8, D_MF), jnp.float32),
            pltpu.VMEM((CH * 8, D_MF), jnp.float32),
            pltpu.VMEM((CH * 8, D_MF), jnp.float32),
            pltpu.VMEM((CH, 128), jnp.float32),
            pltpu.VMEM((CH, 128), jnp.float32),
            pltpu.SemaphoreType.DMA,
            pltpu.SemaphoreType.DMA,
            pltpu.SemaphoreType.DMA,
            pltpu.SemaphoreType.DMA,
        ],
    )
    def k(u_h, i_h, umlp_h, imlp_h, umf_h, imf_h, out_h,
          uv, iv,
          gum0, gum1, gim0, gim1, guf0, guf1, gif0, gif1,
          slab0, slab1,
          sem_g0, sem_g1, sem_s0, sem_s1):
        wid = lax.axis_index("s") * NC + lax.axis_index("c")
        base = wid * BPW
        pltpu.sync_copy(u_h.at[pl.ds(base, BPW)], uv)
        pltpu.sync_copy(i_h.at[pl.ds(base, BPW)], iv)

        sets = ((gum0, gim0, guf0, gif0, slab0, sem_g0, sem_s0),
                (gum1, gim1, guf1, gif1, slab1, sem_g1, sem_s1))

        def fire(p, ua, ia, lo):
            gum, gim, guf, gif, _, sem_g, _ = sets[p]
            for l in range(CH):
                ga = pl.multiple_of(ua[lo + l], 8)
                gb = pl.multiple_of(ia[lo + l], 8)
                sl = pl.ds(l * 8, 8)
                pltpu.async_copy(umlp_h.at[pl.ds(ga, 8)], gum.at[sl], sem_g)
                pltpu.async_copy(imlp_h.at[pl.ds(gb, 8)], gim.at[sl], sem_g)
                pltpu.async_copy(umf_h.at[pl.ds(ga, 8)], guf.at[sl], sem_g)
                pltpu.async_copy(imf_h.at[pl.ds(gb, 8)], gif.at[sl], sem_g)

        def drain_extract(p, us, isub, lo, row0, first):
            gum, gim, guf, gif, slab, sem_g, sem_s = sets[p]
            pltpu.make_async_copy(umlp_h.at[pl.ds(0, CH * 8)], gum, sem_g).wait()
            pltpu.make_async_copy(imlp_h.at[pl.ds(0, CH * 8)], gim, sem_g).wait()
            pltpu.make_async_copy(umf_h.at[pl.ds(0, CH * 8)], guf, sem_g).wait()
            pltpu.make_async_copy(imf_h.at[pl.ds(0, CH * 8)], gif, sem_g).wait()

            @pl.when(jnp.logical_not(first))
            def _():
                pltpu.make_async_copy(
                    slab, out_h.at[pl.ds(pl.multiple_of(base, 8), CH)], sem_s
                ).wait()

            for l in range(CH):
                u8 = us[lo + l]
                i8 = isub[lo + l]
                ru = l * 8 + u8
                ri = l * 8 + i8
                slab[l, pl.ds(0, 16)] = gum[ru, pl.ds(0, 16)]
                slab[l, pl.ds(16, 16)] = gum[ru, pl.ds(16, 16)]
                slab[l, pl.ds(32, 16)] = gim[ri, pl.ds(0, 16)]
                slab[l, pl.ds(48, 16)] = gim[ri, pl.ds(16, 16)]
                slab[l, pl.ds(64, 16)] = guf[ru, pl.ds(0, 16)]
                slab[l, pl.ds(80, 16)] = gif[ri, pl.ds(0, 16)]
            pltpu.async_copy(slab, out_h.at[pl.ds(row0, CH)], sem_s)

        def body(c2, _):
            off = c2 * 16
            uvec = uv[pl.ds(off, 16)]
            ivec = iv[pl.ds(off, 16)]
            ua = uvec - lax.bitwise_and(uvec, 7)
            ia = ivec - lax.bitwise_and(ivec, 7)
            us = lax.bitwise_and(uvec, 7)
            isub = lax.bitwise_and(ivec, 7)
            first = c2 == 0
            row0 = pl.multiple_of(base + off, 8)
            fire(0, ua, ia, 0)
            fire(1, ua, ia, CH)
            drain_extract(0, us, isub, 0, row0, first)
            drain_extract(1, us, isub, CH, row0 + CH, first)
            return 0

        lax.fori_loop(0, NP, body, 0)
        pltpu.make_async_copy(
            slab0, out_h.at[pl.ds(pl.multiple_of(base, 8), CH)], sem_s0
        ).wait()
        pltpu.make_async_copy(
            slab1, out_h.at[pl.ds(pl.multiple_of(base, 8), CH)], sem_s1
        ).wait()

    return k(user_idx, item_idx, U_mlp, I_mlp, U_mf, I_mf)


def _mlp_body(x, w1, b1, w2, b2, w3, b3, wo3, womf, bo, out):
    xb = x[...]
    h = jnp.dot(xb[:, 0:64], w1[...], preferred_element_type=jnp.float32)
    h = jnp.maximum(h + b1[...], 0.0)
    h = jnp.maximum(jnp.dot(h, w2[...], preferred_element_type=jnp.float32) + b2[...], 0.0)
    h = jnp.maximum(jnp.dot(h, w3[...], preferred_element_type=jnp.float32) + b3[...], 0.0)
    z = jnp.dot(h, wo3[...], preferred_element_type=jnp.float32)
    mf = xb[:, 64:80] * xb[:, 80:96]
    z = z + jnp.dot(mf, womf[...], preferred_element_type=jnp.float32)
    out[...] = jax.nn.sigmoid(z + bo[...])


def _tc_mlp(x, W1, b1, W2, b2, W3, b3, Wo, bo):
    wo3 = Wo[:8]
    womf = Wo[8:]
    b1r = b1.reshape(1, -1)
    b2r = b2.reshape(1, -1)
    b3r = b3.reshape(1, -1)
    bor = bo.reshape(1, -1)

    def full(a):
        return pl.BlockSpec(a.shape, lambda i: (0, 0))

    return pl.pallas_call(
        _mlp_body,
        grid=(B // BLK,),
        in_specs=[
            pl.BlockSpec((BLK, 128), lambda i: (i, 0)),
            full(W1), full(b1r), full(W2), full(b2r),
            full(W3), full(b3r), full(wo3), full(womf), full(bor),
        ],
        out_specs=pl.BlockSpec((BLK, 1), lambda i: (i, 0)),
        out_shape=jax.ShapeDtypeStruct((B, 1), jnp.float32),
    )(x, W1, b1r, W2, b2r, W3, b3r, wo3, womf, bor)


def kernel(user_indices, item_indices, U_mlp, I_mlp, U_mf, I_mf,
           W1, b1, W2, b2, W3, b3, Wo, bo):
    x = _sc_gather(user_indices, item_indices, U_mlp, I_mlp, U_mf, I_mf)
    return _tc_mlp(x, W1, b1, W2, b2, W3, b3, Wo, bo)
